# Initial kernel scaffold; baseline (speedup 1.0000x reference)
import jax
import jax.numpy as jnp
from jax.experimental import pallas as pl


def kernel(x, edge_index, batch, W1, A1, W2, A2, M1, b1, M2, b2, M3, b3, M4, b4, M5, b5):
    raise NotImplementedError("write your pallas kernel here")


# speedup vs baseline: 13.5982x; 13.5982x over previous
import functools

import jax
import jax.numpy as jnp
from jax import lax
from jax.experimental import pallas as pl
from jax.experimental.pallas import tpu as pltpu
from jax.experimental.pallas import tpu_sc as plsc

N = 10000
E = 320000
DIN = 128
H = 64
NH = 4
DOUT = 64
G = 64

NC = 2
NS = 16
NW = NC * NS
EW = E // NW
B = 80
NCHUNK = EW // B
ROWS_A = 624
TAIL_OFF = NS * ROWS_A
TAIL = N - TAIL_OFF


def _make_sc_scatter(D):
  mesh = plsc.VectorSubcoreMesh(
      core_axis_name="c", subcore_axis_name="s",
      num_cores=NC, num_subcores=NS)

  @functools.partial(
      pl.kernel,
      out_type=jax.ShapeDtypeStruct((NC, N, D), jnp.float32),
      mesh=mesh,
      scratch_types=[
          pltpu.VMEM((B,), jnp.int32),
          pltpu.VMEM((B,), jnp.int32),
          pltpu.VMEM((B, D), jnp.float32),
          pltpu.VMEM_SHARED((N, D), jnp.float32),
          pltpu.SemaphoreType.DMA,
      ],
  )
  def sc_scatter(x_hbm, row_hbm, col_hbm, zeros_hbm, out_hbm,
                 colv, rowv, gbuf, acc, sem):
    c = lax.axis_index("c")
    s = lax.axis_index("s")
    wid = s * NC + c
    pltpu.sync_copy(zeros_hbm.at[pl.ds(s * ROWS_A, ROWS_A)],
                    acc.at[pl.ds(s * ROWS_A, ROWS_A)])

    @pl.when(s == NS - 1)
    def _():
      pltpu.sync_copy(zeros_hbm.at[pl.ds(TAIL_OFF, TAIL)],
                      acc.at[pl.ds(TAIL_OFF, TAIL)])

    plsc.subcore_barrier()
    base = wid * EW

    @pl.loop(0, NCHUNK)
    def _(i):
      off = base + i * B
      pltpu.sync_copy(col_hbm.at[pl.ds(off, B)], colv)
      pltpu.sync_copy(row_hbm.at[pl.ds(off, B)], rowv)
      pltpu.async_copy(x_hbm.at[colv], gbuf, sem).wait()
      pltpu.sync_copy(gbuf, acc.at[rowv], add=True)

    plsc.subcore_barrier()
    pltpu.sync_copy(acc.at[pl.ds(s * ROWS_A, ROWS_A)],
                    out_hbm.at[c].at[pl.ds(s * ROWS_A, ROWS_A)])

    @pl.when(s == NS - 1)
    def _():
      pltpu.sync_copy(acc.at[pl.ds(TAIL_OFF, TAIL)],
                      out_hbm.at[c].at[pl.ds(TAIL_OFF, TAIL)])

  return sc_scatter


_sc_scatter_128 = _make_sc_scatter(DIN)


BN = 1000


def _dense1_body(g1a, g1b, w1, w2, o):
  g = g1a[...] + g1b[...]
  h1 = lax.dot_general(g, w1[...], (((1,), (1,)), ((), ())),
                       preferred_element_type=jnp.float32)
  h1 = jnp.where(h1 > 0, h1, jnp.exp(jnp.minimum(h1, 0.0)) - 1.0)
  o[...] = lax.dot_general(h1, w2[...], (((1,), (1,)), ((), ())),
                           preferred_element_type=jnp.float32)


def _dense1(g1a, g1b, w1r, w2p):
  return pl.pallas_call(
      _dense1_body,
      grid=(N // BN,),
      in_specs=[
          pl.BlockSpec((BN, DIN), lambda i: (i, 0)),
          pl.BlockSpec((BN, DIN), lambda i: (i, 0)),
          pl.BlockSpec((NH * H, DIN), lambda i: (0, 0)),
          pl.BlockSpec((DIN, NH * H), lambda i: (0, 0)),
      ],
      out_specs=pl.BlockSpec((BN, DIN), lambda i: (i, 0)),
      out_shape=jax.ShapeDtypeStruct((N, DIN), jnp.float32),
  )(g1a, g1b, w1r, w2p)


def _pool_mlp_body(h2a, h2b, batch, m1, b1, m2, b2, m3, b3, m4, b4, m5, b5, o):
  h2 = h2a[...] + h2b[...]
  gids = lax.broadcasted_iota(jnp.int32, (G, N), 0)
  p = (batch[...] == gids).astype(jnp.float32)
  sums = lax.dot_general(p, h2, (((1,), (0,)), ((), ())),
                         preferred_element_type=jnp.float32)
  counts = jnp.sum(p, axis=1, keepdims=True)
  y = (sums / jnp.maximum(counts, 1.0))[:, :DOUT]
  for m, b in ((m1, b1), (m2, b2), (m3, b3), (m4, b4)):
    y = lax.dot_general(y, m[...], (((1,), (1,)), ((), ())),
                        preferred_element_type=jnp.float32) + b[...]
    y = jnp.maximum(y, 0.0)
  logits = lax.dot_general(y, m5[...], (((1,), (1,)), ((), ())),
                           preferred_element_type=jnp.float32) + b5[...]
  z = logits - jnp.max(logits, axis=-1, keepdims=True)
  ez = jnp.exp(z)
  o[...] = ez / jnp.sum(ez, axis=-1, keepdims=True)


def _pool_mlp(h2a, h2b, batch2d, m1, b1, m2, b2, m3, b3, m4, b4, m5, b5):
  return pl.pallas_call(
      _pool_mlp_body,
      out_shape=jax.ShapeDtypeStruct((G, 4), jnp.float32),
  )(h2a, h2b, batch2d, m1, b1, m2, b2, m3, b3, m4, b4, m5, b5)


def kernel(x, edge_index, batch, W1, A1, W2, A2,
           M1, b1, M2, b2, M3, b3, M4, b4, M5, b5):
  row = edge_index[0]
  col = edge_index[1]
  zeros128 = jnp.zeros((N, DIN), jnp.float32)
  g1p = _sc_scatter_128(x, row, col, zeros128)
  w1r = W1.reshape(NH * H, DIN)
  w2p = jnp.zeros((DIN, NH * H), jnp.float32).at[:DOUT].set(W2)
  z2 = _dense1(g1p[0], g1p[1], w1r, w2p)
  h2p = _sc_scatter_128(z2, row, col, zeros128)
  return _pool_mlp(h2p[0], h2p[1], batch.reshape(1, N),
                   M1, b1.reshape(1, -1), M2, b2.reshape(1, -1),
                   M3, b3.reshape(1, -1), M4, b4.reshape(1, -1),
                   M5, b5.reshape(1, -1))


# speedup vs baseline: 21.9102x; 1.6113x over previous
import functools

import jax
import jax.numpy as jnp
from jax import lax
from jax.experimental import pallas as pl
from jax.experimental.pallas import tpu as pltpu
from jax.experimental.pallas import tpu_sc as plsc

N = 10000
E = 320000
DIN = 128
H = 64
NH = 4
DOUT = 64
G = 64

NPAD = 10112

NC = 2
NS = 16
NW = NC * NS
EW = E // NW
B = 80
NCHUNK = EW // B
ROWS_A = 624
TAIL_OFF = NS * ROWS_A
TAIL = N - TAIL_OFF
PAD_ROWS = NPAD - N

NBUF = 4
GROUPS = NCHUNK // NBUF
TAIL_CHUNK = GROUPS * NBUF


def _make_sc_scatter(D):
  mesh = plsc.VectorSubcoreMesh(
      core_axis_name="c", subcore_axis_name="s",
      num_cores=NC, num_subcores=NS)

  @functools.partial(
      pl.kernel,
      out_type=jax.ShapeDtypeStruct((NC, NPAD, D), jnp.float32),
      mesh=mesh,
      scratch_types=[
          [pltpu.VMEM((B,), jnp.int32) for _ in range(NBUF)],
          [pltpu.VMEM((B,), jnp.int32) for _ in range(NBUF)],
          [pltpu.VMEM((B, D), jnp.float32) for _ in range(NBUF)],
          pltpu.VMEM_SHARED((N, D), jnp.float32),
          [pltpu.SemaphoreType.DMA for _ in range(NBUF)],
          [pltpu.SemaphoreType.DMA for _ in range(NBUF)],
      ],
  )
  def sc_scatter(x_hbm, row_hbm, col_hbm, zeros_hbm, out_hbm,
                 colv, rowv, gbuf, acc, sem_i, sem_g):
    c = lax.axis_index("c")
    s = lax.axis_index("s")
    wid = s * NC + c
    pltpu.sync_copy(zeros_hbm.at[pl.ds(s * ROWS_A, ROWS_A)],
                    acc.at[pl.ds(s * ROWS_A, ROWS_A)])

    @pl.when(s == NS - 1)
    def _():
      pltpu.sync_copy(zeros_hbm.at[pl.ds(TAIL_OFF, TAIL)],
                      acc.at[pl.ds(TAIL_OFF, TAIL)])

    plsc.subcore_barrier()
    base = wid * EW

    def start_idx(chunk, b):
      off = base + chunk * B
      pltpu.async_copy(col_hbm.at[pl.ds(off, B)], colv[b], sem_i[b])
      pltpu.async_copy(row_hbm.at[pl.ds(off, B)], rowv[b], sem_i[b])

    def wait_idx(b):
      pltpu.make_async_copy(col_hbm.at[pl.ds(0, B)], colv[b],
                            sem_i[b]).wait()
      pltpu.make_async_copy(row_hbm.at[pl.ds(0, B)], rowv[b],
                            sem_i[b]).wait()

    for b in range(NBUF):
      start_idx(b, b)

    @pl.loop(0, GROUPS)
    def _(g):
      for b in range(NBUF):
        wait_idx(b)
        pltpu.async_copy(x_hbm.at[colv[b]], gbuf[b], sem_g[b])
      for b in range(NBUF):
        pltpu.make_async_copy(x_hbm.at[colv[b]], gbuf[b],
                              sem_g[b]).wait()
        pltpu.sync_copy(gbuf[b], acc.at[rowv[b]], add=True)

        @pl.when(g < GROUPS - 1)
        def _():
          start_idx((g + 1) * NBUF + b, b)

    start_idx(TAIL_CHUNK, 0)
    wait_idx(0)
    pltpu.async_copy(x_hbm.at[colv[0]], gbuf[0], sem_g[0]).wait()
    pltpu.sync_copy(gbuf[0], acc.at[rowv[0]], add=True)

    plsc.subcore_barrier()
    pltpu.sync_copy(acc.at[pl.ds(s * ROWS_A, ROWS_A)],
                    out_hbm.at[c].at[pl.ds(s * ROWS_A, ROWS_A)])

    @pl.when(s == NS - 1)
    def _():
      pltpu.sync_copy(acc.at[pl.ds(TAIL_OFF, TAIL)],
                      out_hbm.at[c].at[pl.ds(TAIL_OFF, TAIL)])
      pltpu.sync_copy(zeros_hbm.at[pl.ds(0, PAD_ROWS)],
                      out_hbm.at[c].at[pl.ds(N, PAD_ROWS)])

  return sc_scatter


_sc_scatter_128 = _make_sc_scatter(DIN)

GN = G * NPAD
CSEG = GN // NS


def _make_sc_cmat():
  mesh = plsc.VectorSubcoreMesh(
      core_axis_name="c", subcore_axis_name="s",
      num_cores=NC, num_subcores=NS)

  @functools.partial(
      pl.kernel,
      out_type=jax.ShapeDtypeStruct((NC, GN), jnp.float32),
      mesh=mesh,
      scratch_types=[
          [pltpu.VMEM((B,), jnp.int32) for _ in range(NBUF)],
          [pltpu.VMEM((B,), jnp.int32) for _ in range(NBUF)],
          [pltpu.VMEM((B,), jnp.int32) for _ in range(NBUF)],
          pltpu.VMEM((B,), jnp.float32),
          pltpu.VMEM_SHARED((GN,), jnp.float32),
          [pltpu.SemaphoreType.DMA for _ in range(NBUF)],
          [pltpu.SemaphoreType.DMA for _ in range(NBUF)],
      ],
  )
  def sc_cmat(row_hbm, col_hbm, batch_hbm, zeros_hbm, out_hbm,
              colv, rowv, browv, ones, acc, sem_i, sem_g):
    c = lax.axis_index("c")
    s = lax.axis_index("s")
    wid = s * NC + c
    for j in range(B // 16):
      ones[pl.ds(j * 16, 16)] = jnp.full((16,), 1.0, jnp.float32)
    pltpu.sync_copy(zeros_hbm.at[pl.ds(s * CSEG, CSEG)],
                    acc.at[pl.ds(s * CSEG, CSEG)])
    plsc.subcore_barrier()
    base = wid * EW

    def start_idx(chunk, b):
      off = base + chunk * B
      pltpu.async_copy(col_hbm.at[pl.ds(off, B)], colv[b], sem_i[b])
      pltpu.async_copy(row_hbm.at[pl.ds(off, B)], rowv[b], sem_i[b])

    def wait_idx(b):
      pltpu.make_async_copy(col_hbm.at[pl.ds(0, B)], colv[b],
                            sem_i[b]).wait()
      pltpu.make_async_copy(row_hbm.at[pl.ds(0, B)], rowv[b],
                            sem_i[b]).wait()

    def do_chunk(b):
      pltpu.make_async_copy(batch_hbm.at[rowv[b]], browv[b],
                            sem_g[b]).wait()
      for j in range(B // 16):
        sl = pl.ds(j * 16, 16)
        colv[b][sl] = browv[b][sl] * NPAD + colv[b][sl]
      pltpu.sync_copy(ones, acc.at[colv[b]], add=True)

    for b in range(NBUF):
      start_idx(b, b)

    @pl.loop(0, GROUPS)
    def _(g):
      for b in range(NBUF):
        wait_idx(b)
        pltpu.async_copy(batch_hbm.at[rowv[b]], browv[b], sem_g[b])
      for b in range(NBUF):
        do_chunk(b)

        @pl.when(g < GROUPS - 1)
        def _():
          start_idx((g + 1) * NBUF + b, b)

    start_idx(TAIL_CHUNK, 0)
    wait_idx(0)
    pltpu.async_copy(batch_hbm.at[rowv[0]], browv[0], sem_g[0])
    do_chunk(0)

    plsc.subcore_barrier()
    pltpu.sync_copy(acc.at[pl.ds(s * CSEG, CSEG)],
                    out_hbm.at[c].at[pl.ds(s * CSEG, CSEG)])

  return sc_cmat


_sc_cmat = _make_sc_cmat()


BN = 1264


_HI = lax.Precision.HIGHEST
BN0 = 1000


def _dense0_body(x, w1a, w1b, oa, ob):
  xb = x[...].astype(jnp.bfloat16)
  oa[...] = lax.dot_general(xb, w1a[...], (((1,), (1,)), ((), ())),
                            preferred_element_type=jnp.float32)
  ob[...] = lax.dot_general(xb, w1b[...], (((1,), (1,)), ((), ())),
                            preferred_element_type=jnp.float32)


def _dense0(x, w1a, w1b):
  return pl.pallas_call(
      _dense0_body,
      grid=(N // BN0,),
      in_specs=[
          pl.BlockSpec((BN0, DIN), lambda i: (i, 0)),
          pl.BlockSpec((DIN, DIN), lambda i: (0, 0)),
          pl.BlockSpec((DIN, DIN), lambda i: (0, 0)),
      ],
      out_specs=[pl.BlockSpec((BN0, DIN), lambda i: (i, 0)),
                 pl.BlockSpec((BN0, DIN), lambda i: (i, 0))],
      out_shape=[jax.ShapeDtypeStruct((N, DIN), jnp.float32),
                 jax.ShapeDtypeStruct((N, DIN), jnp.float32)],
  )(x, w1a, w1b)


def _dense1_body(ga0, ga1, gb0, gb1, w2a, w2b, o):
  def elu(v):
    return jnp.where(v > 0, v, jnp.exp(jnp.minimum(v, 0.0)) - 1.0)
  h1a = elu(ga0[...] + ga1[...]).astype(jnp.bfloat16)
  h1b = elu(gb0[...] + gb1[...]).astype(jnp.bfloat16)
  o[...] = (
      lax.dot_general(h1a, w2a[...], (((1,), (1,)), ((), ())),
                      preferred_element_type=jnp.float32)
      + lax.dot_general(h1b, w2b[...], (((1,), (1,)), ((), ())),
                        preferred_element_type=jnp.float32))


def _dense1(ga0, ga1, gb0, gb1, w2a, w2b):
  return pl.pallas_call(
      _dense1_body,
      grid=(NPAD // BN,),
      in_specs=[pl.BlockSpec((BN, DIN), lambda i: (i, 0))] * 4
      + [pl.BlockSpec((DIN, DIN), lambda i: (0, 0))] * 2,
      out_specs=pl.BlockSpec((BN, DIN), lambda i: (i, 0)),
      out_shape=jax.ShapeDtypeStruct((NPAD, DIN), jnp.float32),
  )(ga0, ga1, gb0, gb1, w2a, w2b)


def _pool_mlp_body(c0, c1, z2, batch, m1, b1, m2, b2, m3, b3, m4, b4,
                   m5, b5, o):
  cm = c0[...] + c1[...]
  sums = lax.dot_general(cm, z2[...], (((1,), (0,)), ((), ())),
                         precision=_HI, preferred_element_type=jnp.float32)
  gids = lax.broadcasted_iota(jnp.int32, (G, NPAD), 0)
  counts = jnp.sum((batch[...] == gids).astype(jnp.float32),
                   axis=1, keepdims=True)
  y = sums / jnp.maximum(counts, 1.0)
  for m, b in ((m1, b1), (m2, b2), (m3, b3), (m4, b4)):
    y = lax.dot_general(y.astype(jnp.bfloat16), m[...],
                        (((1,), (1,)), ((), ())),
                        preferred_element_type=jnp.float32)
    y = jnp.maximum(y + b[...], 0.0)
  logits = lax.dot_general(y.astype(jnp.bfloat16), m5[...],
                           (((1,), (1,)), ((), ())),
                           preferred_element_type=jnp.float32) + b5[...]
  z = logits - jnp.max(logits, axis=-1, keepdims=True)
  ez = jnp.exp(z)
  o[...] = (ez / jnp.sum(ez, axis=-1, keepdims=True))[:, :4]


def _pool_mlp(c0, c1, z2, batch2d, m1, b1, m2, b2, m3, b3, m4, b4, m5, b5):
  return pl.pallas_call(
      _pool_mlp_body,
      out_shape=jax.ShapeDtypeStruct((G, 4), jnp.float32),
  )(c0, c1, z2, batch2d, m1, b1, m2, b2, m3, b3, m4, b4, m5, b5)


def kernel(x, edge_index, batch, W1, A1, W2, A2,
           M1, b1, M2, b2, M3, b3, M4, b4, M5, b5):
  row = edge_index[0]
  col = edge_index[1]
  zeros128 = jnp.zeros((N, DIN), jnp.float32)
  zerosgn = jnp.zeros((GN,), jnp.float32)
  batch_pad = jnp.full((1, NPAD), -1, jnp.int32).at[:, :N].set(
      batch.reshape(1, N))
  bfc = jnp.bfloat16
  w1r = W1.reshape(NH * H, DIN)
  z1a, z1b = _dense0(x, w1r[:DIN].astype(bfc), w1r[DIN:].astype(bfc))
  g1pa = _sc_scatter_128(z1a, row, col, zeros128)
  g1pb = _sc_scatter_128(z1b, row, col, zeros128)
  cpart = _sc_cmat(row, col, batch, zerosgn)
  w2p = jnp.zeros((DIN, NH * H), jnp.float32).at[:DOUT].set(W2)
  z2 = _dense1(g1pa[0], g1pa[1], g1pb[0], g1pb[1],
               w2p[:, :DIN].astype(bfc), w2p[:, DIN:].astype(bfc))
  cp = cpart.reshape(NC, G, NPAD)
  m1p = jnp.zeros((1024, DIN), jnp.float32).at[:, :DOUT].set(M1).astype(bfc)
  m5p = jnp.zeros((DIN, 1024), jnp.float32).at[:4].set(M5).astype(bfc)
  b5p = jnp.full((1, DIN), -1e30, jnp.float32).at[:, :4].set(
      b5.reshape(1, -1))
  return _pool_mlp(cp[0], cp[1], z2, batch_pad,
                   m1p, b1.reshape(1, -1), M2.astype(bfc),
                   b2.reshape(1, -1), M3.astype(bfc), b3.reshape(1, -1),
                   M4.astype(bfc), b4.reshape(1, -1), m5p, b5p)
